# baseline (speedup 1.0000x reference)
import functools

import jax
import jax.numpy as jnp
from jax import lax
from jax.experimental import pallas as pl
from jax.experimental.pallas import tpu as pltpu
from jax.experimental.pallas import tpu_sc as plsc

BATCH, SEQ, VOCAB, DIM = 4096, 200, 1000000, 32
NC, NS = 2, 16
NW = NC * NS
RPW = BATCH // NW
LANES = 16
NBUF = 8
AHEAD = 4
H0, H1 = 104, 96
SLOTS = RPW + 8


def _body(ids_hbm, mask_hbm, table_hbm, out_hbm, ids_v, mask_v, rows_v,
          sidx_a, sidx_b, res_v, acc_sh, *sems):
    gsem = sems[:NBUF]
    ssem = sems[NBUF:]
    sid = lax.axis_index("s")
    wid = sid * NC + lax.axis_index("c")
    base = wid * RPW
    slot0 = sid * SLOTS
    trash_v = jnp.full((LANES,), slot0 + RPW, jnp.int32)

    def zrow(r, _):
        res_v[r, pl.ds(0, LANES)] = jnp.zeros((LANES,), jnp.float32)
        res_v[r, pl.ds(LANES, LANES)] = jnp.zeros((LANES,), jnp.float32)
        return 0

    lax.fori_loop(0, RPW, zrow, 0)
    pltpu.sync_copy(res_v, acc_sh.at[pl.ds(slot0, RPW)])
    pltpu.sync_copy(res_v.at[pl.ds(0, SLOTS - RPW)],
                    acc_sh.at[pl.ds(slot0 + RPW, SLOTS - RPW)])

    pltpu.sync_copy(ids_hbm.at[pl.ds(base, RPW)], ids_v)
    pltpu.sync_copy(mask_hbm.at[pl.ds(base, RPW)], mask_v)

    def fire_gather(j, b):
        pltpu.async_copy(table_hbm.at[ids_v.at[j, pl.ds(0, H0)]],
                         rows_v.at[b, pl.ds(0, H0)], gsem[b])
        pltpu.async_copy(table_hbm.at[ids_v.at[j, pl.ds(H0, H1)]],
                         rows_v.at[b, pl.ds(H0, H1)], gsem[b])

    def wait_gather(j, b):
        pltpu.make_async_copy(table_hbm.at[ids_v.at[j, pl.ds(0, H0)]],
                              rows_v.at[b, pl.ds(0, H0)], gsem[b]).wait()
        pltpu.make_async_copy(table_hbm.at[ids_v.at[j, pl.ds(H0, H1)]],
                              rows_v.at[b, pl.ds(H0, H1)],
                              gsem[b]).wait()

    def fire_scatter(b):
        pltpu.async_copy(rows_v.at[b, pl.ds(0, H0)],
                         acc_sh.at[sidx_a.at[b]], ssem[b], add=True)
        pltpu.async_copy(rows_v.at[b, pl.ds(H0, H1)],
                         acc_sh.at[sidx_b.at[b]], ssem[b], add=True)

    def wait_scatter(b):
        pltpu.make_async_copy(rows_v.at[b, pl.ds(0, H0)],
                              acc_sh.at[sidx_a.at[b]], ssem[b]).wait()
        pltpu.make_async_copy(rows_v.at[b, pl.ds(H0, H1)],
                              acc_sh.at[sidx_b.at[b]],
                              ssem[b]).wait()

    def build_sidx(j, b):
        slot_v = jnp.full((LANES,), slot0 + j, jnp.int32)
        for c in range(7):
            o = min(c * LANES, H0 - LANES)
            m = mask_v[j, pl.ds(o, LANES)]
            sidx_a[b, pl.ds(o, LANES)] = jnp.where(m > 0, slot_v, trash_v)
        for c in range(6):
            m = mask_v[j, pl.ds(H0 + c * LANES, LANES)]
            sidx_b[b, pl.ds(c * LANES, LANES)] = jnp.where(
                m > 0, slot_v, trash_v)

    for b in range(AHEAD):
        fire_gather(b, b)

    def ring(q, _):
        for b in range(NBUF):
            i = q * NBUF + b
            build_sidx(i, b)
            wait_gather(i, b)
            fire_scatter(b)
            r = i + AHEAD
            br = (b + AHEAD) % NBUF

            @pl.when(r < RPW)
            def _():
                @pl.when(i >= AHEAD)
                def _():
                    wait_scatter(br)

                fire_gather(r, br)
        return 0

    lax.fori_loop(0, RPW // NBUF, ring, 0)
    for b in range(NBUF):
        wait_scatter(b)

    pltpu.sync_copy(acc_sh.at[pl.ds(slot0, RPW)], res_v)

    def div_row(j, _):
        zi = jnp.zeros((LANES,), jnp.int32)
        def chunk(c, cnt16):
            return cnt16 + mask_v[j, pl.ds(c * LANES, LANES)]
        cnt16 = lax.fori_loop(0, SEQ // LANES, chunk, zi)
        tail = mask_v[j, pl.ds(SEQ - LANES, LANES)]
        lane = lax.iota(jnp.int32, LANES)
        cnt16 = cnt16 + jnp.where(lane >= LANES // 2, tail, 0)
        cnt = cnt16[0]
        for t in range(1, LANES):
            cnt = cnt + cnt16[t]
        inv = 1.0 / jnp.full((LANES,), cnt.astype(jnp.float32))
        res_v[j, pl.ds(0, LANES)] = res_v[j, pl.ds(0, LANES)] * inv
        res_v[j, pl.ds(LANES, LANES)] = (
            res_v[j, pl.ds(LANES, LANES)] * inv)
        return 0

    lax.fori_loop(0, RPW, div_row, 0)
    pltpu.sync_copy(res_v, out_hbm.at[pl.ds(base, RPW)])


@functools.partial(
    pl.kernel,
    out_type=jax.ShapeDtypeStruct((BATCH, DIM), jnp.float32),
    mesh=plsc.VectorSubcoreMesh(core_axis_name="c", subcore_axis_name="s",
                                num_cores=NC, num_subcores=NS),
    compiler_params=pltpu.CompilerParams(use_tc_tiling_on_sc=False),
    scratch_types=[
        pltpu.VMEM((RPW, SEQ), jnp.int32),
        pltpu.VMEM((RPW, SEQ), jnp.int32),
        pltpu.VMEM((NBUF, SEQ, DIM), jnp.float32),
        pltpu.VMEM((NBUF, H0), jnp.int32),
        pltpu.VMEM((NBUF, H1), jnp.int32),
        pltpu.VMEM((RPW, DIM), jnp.float32),
        pltpu.VMEM_SHARED((NS * SLOTS, DIM), jnp.float32),
    ] + [pltpu.SemaphoreType.DMA] * (2 * NBUF),
)
def _encode(ids_hbm, mask_hbm, table_hbm, out_hbm, *refs):
    _body(ids_hbm, mask_hbm, table_hbm, out_hbm, *refs)


def kernel(input_ids, attention_mask, embedding_table):
    return _encode(input_ids, attention_mask, embedding_table)
